# baseline (speedup 1.0000x reference)
import functools

import jax
import jax.numpy as jnp
import numpy as np
from jax import lax
from jax.experimental import pallas as pl
from jax.experimental.pallas import tpu as pltpu
from jax.experimental.pallas import tpu_sc as plsc

_N = 10000
_NN = 2 * _N
_NNP = 20096
_E = 320000
_E2 = 2 * _E
_DIN = 128
_DHID = 128
_DOUT = 64

_K = 128
_NBUF = 4
_NSBUF = 2
_SBC = 32
_TILES = 16
_CHUNKS = 320
_NSB = _CHUNKS // _SBC
_EPT = _CHUNKS * _K
_EPAD = _EPT * _TILES
_EPADC = _EPAD // _K

_SELU_ALPHA = 1.6732632423543772
_SELU_SCALE = 1.0507009873554805

_R = 400

_T32 = np.concatenate([np.arange(0, 32, 2), np.arange(1, 32, 2)])
_PERM128 = jnp.asarray(np.concatenate([g * 32 + _T32 for g in range(4)]))
_PERM64 = jnp.asarray(np.concatenate([g * 32 + _T32 for g in range(2)]))


def _mm1(x, w):
    h = _DHID // 2

    def body(x_ref, w_ref, o_ref):
        z = jnp.dot(x_ref[...], w_ref[...], preferred_element_type=jnp.float32)
        z = z.astype(jnp.bfloat16)
        o_ref[0] = z[:, :h]
        o_ref[1] = z[:, h:]

    return pl.pallas_call(
        body,
        grid=(_NN // _R,),
        in_specs=[
            pl.BlockSpec((_R, _DIN), lambda i: (i, 0)),
            pl.BlockSpec((_DIN, _DHID), lambda i: (0, 0)),
        ],
        out_specs=pl.BlockSpec((2, _R, h), lambda i: (0, i, 0)),
        out_shape=jax.ShapeDtypeStruct((2, _NN, h), jnp.bfloat16),
    )(x, w)


def _mid(agg, b1p, w2p):
    h = _DHID // 2
    o = _DOUT // 2

    def body(a_ref, b_ref, w_ref, o_ref):
        z = jnp.concatenate([a_ref[0], a_ref[1]], axis=1) + b_ref[...]
        z = _SELU_SCALE * jnp.where(z > 0, z, _SELU_ALPHA * (jnp.exp(z) - 1.0))
        z = jnp.dot(z, w_ref[...], preferred_element_type=jnp.float32)
        z = z.astype(jnp.bfloat16)
        o_ref[0] = z[:, :o]
        o_ref[1] = z[:, o:]

    return pl.pallas_call(
        body,
        grid=(_NN // _R,),
        in_specs=[
            pl.BlockSpec((2, _R, h), lambda i: (0, i, 0)),
            pl.BlockSpec((1, _DHID), lambda i: (0, 0)),
            pl.BlockSpec((_DHID, _DOUT), lambda i: (0, 0)),
        ],
        out_specs=pl.BlockSpec((2, _R, o), lambda i: (0, i, 0)),
        out_shape=jax.ShapeDtypeStruct((2, _NN, o), jnp.bfloat16),
    )(agg, b1p, w2p)


def _final(agg, b2p):
    o = _DOUT // 2

    def fix(q):
        return jnp.stack([q[:, :16], q[:, 16:]], axis=2).reshape(_R, 32)

    def body(a_ref, b_ref, o_ref):
        z = jnp.concatenate([a_ref[0], a_ref[1]], axis=1) + b_ref[...]
        n = jnp.sum(z * z, axis=1, keepdims=True)
        z = z * lax.rsqrt(jnp.maximum(n, 1e-24))
        o_ref[...] = jnp.concatenate([fix(z[:, :32]), fix(z[:, 32:])], axis=1)

    return pl.pallas_call(
        body,
        grid=(_NN // _R,),
        in_specs=[
            pl.BlockSpec((2, _R, o), lambda i: (0, i, 0)),
            pl.BlockSpec((1, _DOUT), lambda i: (0, 0)),
        ],
        out_specs=pl.BlockSpec((_R, _DOUT), lambda i: (i, 0)),
        out_shape=jax.ShapeDtypeStruct((_NN, _DOUT), jnp.float32),
    )(agg, b2p)


def _make_spmm(dh):
    mesh = plsc.VectorSubcoreMesh(
        core_axis_name="c", subcore_axis_name="s", num_cores=2, num_subcores=16
    )
    stripe = _NNP // _TILES

    @functools.partial(
        pl.kernel,
        out_type=jax.ShapeDtypeStruct((2 * _NNP, dh), jnp.float32),
        mesh=mesh,
        scratch_types=[
            pltpu.VMEM_SHARED((_NNP, dh), jnp.float32),
            pltpu.VMEM((_SBC, _K), jnp.int32),
            pltpu.VMEM((_SBC, _K), jnp.int32),
            pltpu.VMEM((_SBC * _K,), jnp.float32),
            pltpu.VMEM((_NBUF, _K, dh), jnp.bfloat16),
            pltpu.VMEM((_NSBUF, _K, dh), jnp.float32),
            [pltpu.SemaphoreType.DMA] * _NBUF,
            [pltpu.SemaphoreType.DMA] * _NSBUF,
        ],
        compiler_params=pltpu.CompilerParams(use_tc_tiling_on_sc=False, needs_layout_passes=False),
    )
    def spmm(table, rows_hbm, cols_hbm, w_hbm, out,
             acc, colv, rowv, wv, gbuf, sbuf, gsems, ssems):
        c = lax.axis_index("c")
        s = lax.axis_index("s")

        zero = jnp.zeros((16,), jnp.float32)

        def z_body(r, carry):
            for d in range(dh // 16):
                sbuf[0, r, pl.ds(d * 16, 16)] = zero
            return carry

        lax.fori_loop(0, _K, z_body, 0)
        r0 = s * stripe
        for i in range(stripe // _K):
            pltpu.sync_copy(sbuf.at[0], acc.at[pl.ds(r0 + i * _K, _K)])
        rem = stripe % _K
        pltpu.sync_copy(sbuf.at[0, pl.ds(0, rem)],
                        acc.at[pl.ds(r0 + (stripe // _K) * _K, rem)])
        plsc.subcore_barrier()

        def gstart(b, k):
            pltpu.async_copy(table.at[colv.at[k]], gbuf.at[b], gsems[b])

        def gwait(b):
            pltpu.make_async_copy(table.at[colv.at[0]], gbuf.at[b],
                                  gsems[b]).wait()

        def sstart(m, k):
            pltpu.async_copy(sbuf.at[m], acc.at[rowv.at[k]], ssems[m],
                             add=True)

        def swait(m):
            pltpu.make_async_copy(out.at[pl.ds(0, _K)], sbuf.at[m],
                                  ssems[m]).wait()

        lane_dnums = lax.GatherDimensionNumbers(
            offset_dims=(), collapsed_slice_dims=(0,), start_index_map=(0,))
        lane_idx = [jnp.full((16, 1), u, jnp.int32) for u in range(16)]

        def bcast_lane(vec, u):
            return lax.gather(vec, lane_idx[u], lane_dnums, (1,),
                              mode=lax.GatherScatterMode.PROMISE_IN_BOUNDS)

        def scale(b, m, k):
            wbase = k * _K

            def g_body(g, carry):
                w16 = wv[pl.ds(wbase + g * 16, 16)]
                for u in range(16):
                    wb = bcast_lane(w16, u)
                    e = g * 16 + u
                    for d in range(dh // 32):
                        ab = gbuf[b, e, pl.ds(d * 32, 32)]
                        va, vb = plsc.unpack(
                            ab, format=plsc.PackFormat.INTERLEAVED)
                        sbuf[m, e, pl.ds(d * 32, 16)] = va * wb
                        sbuf[m, e, pl.ds(d * 32 + 16, 16)] = vb * wb
                return carry

            lax.fori_loop(0, _K // 16, g_body, 0)

        cbase = s * _CHUNKS

        def sb_body(sb, carry):
            cb = cbase + sb * _SBC
            pltpu.sync_copy(cols_hbm.at[c, pl.ds(cb, _SBC)], colv)
            pltpu.sync_copy(rows_hbm.at[pl.ds(cb, _SBC)], rowv)
            pltpu.sync_copy(w_hbm.at[pl.ds(cb * _K, _SBC * _K)], wv)
            for g in range(_NBUF - 1):
                gstart(g, g)

            def quad_body(k4, carry2):
                for u in range(_NBUF):
                    g = k4 * _NBUF + u
                    m = u % _NSBUF
                    gwait(u)

                    @pl.when(g >= _NSBUF)
                    def _():
                        swait(m)

                    scale(u, m, g)
                    sstart(m, g)
                    nb = (u + _NBUF - 1) % _NBUF

                    @pl.when(g + _NBUF - 1 < _SBC)
                    def _():
                        gstart(nb, g + _NBUF - 1)
                return carry2

            lax.fori_loop(0, _SBC // _NBUF, quad_body, 0)
            for m in range(_NSBUF):
                swait(m)
            return carry

        lax.fori_loop(0, _NSB, sb_body, 0)

        plsc.subcore_barrier()
        pltpu.sync_copy(acc.at[pl.ds(r0, stripe)],
                        out.at[pl.ds(c * _NNP + r0, stripe)])

    return spmm


_spmm_hid = _make_spmm(_DHID // 2)
_spmm_out = _make_spmm(_DOUT // 2)


def kernel(edge_index1, edge_weight1, edge_index2, edge_weight2,
           X1, X2, W1, b1, W2, b2):
    x = jnp.concatenate([X1, X2], axis=0)
    row = jnp.concatenate([edge_index1[0], edge_index2[0] + _N])
    col = jnp.concatenate([edge_index1[1], edge_index2[1] + _N])
    w = jnp.concatenate([edge_weight1, edge_weight2])
    pad = _EPAD - _E2
    row = jnp.pad(row, (0, pad)).reshape(_EPADC, _K)
    colp = jnp.pad(col, (0, pad))
    cols = jnp.stack([colp, colp + _NN]).reshape(2, _EPADC, _K)
    w = jnp.pad(w, (0, pad))

    b1p = jnp.take(b1, _PERM128).reshape(1, _DHID)
    w2p = jnp.take(W2, _PERM128, axis=0)
    b2p = jnp.take(b2, _PERM64).reshape(1, _DOUT)

    z = _mm1(x, W1)
    z = _spmm_hid(z.reshape(2 * _NN, _DHID // 2), row, cols, w)
    z = z.reshape(2, _NNP, _DHID // 2)[:, :_NN, :]
    z = _mid(z, b1p, w2p)
    z = _spmm_out(z.reshape(2 * _NN, _DOUT // 2), row, cols, w)
    z = z.reshape(2, _NNP, _DOUT // 2)[:, :_NN, :]
    z = _final(z, b2p)
    return z[:_N], z[_N:]


# speedup vs baseline: 1.2779x; 1.2779x over previous
import functools

import jax
import jax.numpy as jnp
import numpy as np
from jax import lax
from jax.experimental import pallas as pl
from jax.experimental.pallas import tpu as pltpu
from jax.experimental.pallas import tpu_sc as plsc

_N = 10000
_NN = 2 * _N
_NNP = 20096
_E = 320000
_E2 = 2 * _E
_DIN = 128
_DHID = 128
_DOUT = 64

_K = 128
_NBUF = 4
_NSBUF = 2
_SBC = 32
_TILES = 16
_CHUNKS = 320
_NSB = _CHUNKS // _SBC
_EPT = _CHUNKS * _K
_EPAD = _EPT * _TILES
_EPADC = _EPAD // _K

_SELU_ALPHA = 1.6732632423543772
_SELU_SCALE = 1.0507009873554805

_R = 800

_QG = np.ravel(np.stack([np.arange(16), np.arange(16) + 16], axis=1))
_Q128 = jnp.asarray(np.concatenate([b + _QG for b in range(0, 128, 32)]))
_Q64 = jnp.asarray(np.concatenate([b + _QG for b in range(0, 64, 32)]))


def _mm1(x, w1q):
    h = _DHID // 2

    def body(x_ref, w_ref, o_ref):
        z = jnp.dot(x_ref[...], w_ref[...], preferred_element_type=jnp.float32)
        z = z.astype(jnp.bfloat16)
        o_ref[0] = z[:, :h]
        o_ref[1] = z[:, h:]

    return pl.pallas_call(
        body,
        grid=(_NN // _R,),
        in_specs=[
            pl.BlockSpec((_R, _DIN), lambda i: (i, 0)),
            pl.BlockSpec((_DIN, _DHID), lambda i: (0, 0)),
        ],
        out_specs=pl.BlockSpec((2, _R, h), lambda i: (0, i, 0)),
        out_shape=jax.ShapeDtypeStruct((2, _NN, h), jnp.bfloat16),
    )(x, w1q)


def _mid(agg, b1, w2q):
    h = _DHID // 2
    o = _DOUT // 2

    def body(a_ref, b_ref, w_ref, o_ref):
        z = jnp.concatenate([a_ref[0], a_ref[1]], axis=1) + b_ref[...]
        z = _SELU_SCALE * jnp.where(z > 0, z, _SELU_ALPHA * (jnp.exp(z) - 1.0))
        z = jnp.dot(z, w_ref[...], preferred_element_type=jnp.float32)
        z = z.astype(jnp.bfloat16)
        o_ref[0] = z[:, :o]
        o_ref[1] = z[:, o:]

    return pl.pallas_call(
        body,
        grid=(_NN // _R,),
        in_specs=[
            pl.BlockSpec((2, _R, h), lambda i: (0, i, 0)),
            pl.BlockSpec((1, _DHID), lambda i: (0, 0)),
            pl.BlockSpec((_DHID, _DOUT), lambda i: (0, 0)),
        ],
        out_specs=pl.BlockSpec((2, _R, o), lambda i: (0, i, 0)),
        out_shape=jax.ShapeDtypeStruct((2, _NN, o), jnp.bfloat16),
    )(agg, b1, w2q)


def _final(agg, b2):
    o = _DOUT // 2

    def body(a_ref, b_ref, o_ref):
        z = jnp.concatenate([a_ref[0], a_ref[1]], axis=1) + b_ref[...]
        n = jnp.sum(z * z, axis=1, keepdims=True)
        o_ref[...] = z * lax.rsqrt(jnp.maximum(n, 1e-24))

    return pl.pallas_call(
        body,
        grid=(_NN // _R,),
        in_specs=[
            pl.BlockSpec((2, _R, o), lambda i: (0, i, 0)),
            pl.BlockSpec((1, _DOUT), lambda i: (0, 0)),
        ],
        out_specs=pl.BlockSpec((_R, _DOUT), lambda i: (i, 0)),
        out_shape=jax.ShapeDtypeStruct((_NN, _DOUT), jnp.float32),
    )(agg, b2)


def _make_spmm(dh):
    mesh = plsc.VectorSubcoreMesh(
        core_axis_name="c", subcore_axis_name="s", num_cores=2, num_subcores=16
    )
    stripe = _NNP // _TILES
    last = _NN - 15 * stripe

    @functools.partial(
        pl.kernel,
        out_type=jax.ShapeDtypeStruct((2 * _NN, dh), jnp.float32),
        mesh=mesh,
        scratch_types=[
            pltpu.VMEM_SHARED((_NNP, dh), jnp.float32),
            pltpu.VMEM((_SBC, _K), jnp.int32),
            pltpu.VMEM((_SBC, _K), jnp.int32),
            pltpu.VMEM((_SBC * _K,), jnp.float32),
            pltpu.VMEM((_NBUF, _K, dh), jnp.bfloat16),
            pltpu.VMEM((_NSBUF, _K, dh), jnp.float32),
            [pltpu.SemaphoreType.DMA] * _NBUF,
            [pltpu.SemaphoreType.DMA] * _NSBUF,
        ],
        compiler_params=pltpu.CompilerParams(
            use_tc_tiling_on_sc=False, needs_layout_passes=False),
    )
    def spmm(table, rows_hbm, cols_hbm, w_hbm, out,
             acc, colv, rowv, wv, gbuf, sbuf, gsems, ssems):
        c = lax.axis_index("c")
        s = lax.axis_index("s")

        zero = jnp.zeros((16,), jnp.float32)

        def z_body(r, carry):
            for d in range(dh // 16):
                sbuf[0, r, pl.ds(d * 16, 16)] = zero
            return carry

        lax.fori_loop(0, _K, z_body, 0)
        r0 = s * stripe
        for i in range(stripe // _K):
            pltpu.sync_copy(sbuf.at[0], acc.at[pl.ds(r0 + i * _K, _K)])
        rem = stripe % _K
        pltpu.sync_copy(sbuf.at[0, pl.ds(0, rem)],
                        acc.at[pl.ds(r0 + (stripe // _K) * _K, rem)])
        plsc.subcore_barrier()

        def gstart(b, k):
            pltpu.async_copy(table.at[colv.at[k]], gbuf.at[b], gsems[b])

        def gwait(b):
            pltpu.make_async_copy(table.at[colv.at[0]], gbuf.at[b],
                                  gsems[b]).wait()

        def sstart(m, k):
            pltpu.async_copy(sbuf.at[m], acc.at[rowv.at[k]], ssems[m],
                             add=True)

        def swait(m):
            pltpu.make_async_copy(out.at[pl.ds(0, _K)], sbuf.at[m],
                                  ssems[m]).wait()

        lane_dnums = lax.GatherDimensionNumbers(
            offset_dims=(), collapsed_slice_dims=(0,), start_index_map=(0,))
        lane_idx = [jnp.full((16, 1), u, jnp.int32) for u in range(16)]

        def bcast_lane(vec, u):
            return lax.gather(vec, lane_idx[u], lane_dnums, (1,),
                              mode=lax.GatherScatterMode.PROMISE_IN_BOUNDS)

        def scale(b, m, k):
            wbase = k * _K

            def g_body(g, carry):
                w16 = wv[pl.ds(wbase + g * 16, 16)]
                for u in range(16):
                    wb = bcast_lane(w16, u)
                    e = g * 16 + u
                    for d in range(dh // 32):
                        ab = gbuf[b, e, pl.ds(d * 32, 32)]
                        va, vb = plsc.unpack(
                            ab, format=plsc.PackFormat.INTERLEAVED)
                        sbuf[m, e, pl.ds(d * 32, 16)] = va * wb
                        sbuf[m, e, pl.ds(d * 32 + 16, 16)] = vb * wb
                return carry

            lax.fori_loop(0, _K // 16, g_body, 0)

        cbase = s * _CHUNKS
        coff = jnp.broadcast_to(c * _NN, (16,)).astype(jnp.int32)

        def sb_body(sb, carry):
            cb = cbase + sb * _SBC
            pltpu.sync_copy(cols_hbm.at[pl.ds(cb, _SBC)], colv)
            pltpu.sync_copy(rows_hbm.at[pl.ds(cb, _SBC)], rowv)
            pltpu.sync_copy(w_hbm.at[pl.ds(cb * _K, _SBC * _K)], wv)

            @pl.when(c > 0)
            def _():
                def adj_body(j, carry2):
                    for d in range(_K // 16):
                        sl = pl.ds(d * 16, 16)
                        colv[j, sl] = colv[j, sl] + coff
                    return carry2

                lax.fori_loop(0, _SBC, adj_body, 0)

            for g in range(_NBUF - 1):
                gstart(g, g)

            def quad_body(k4, carry2):
                for u in range(_NBUF):
                    g = k4 * _NBUF + u
                    m = u % _NSBUF
                    gwait(u)

                    @pl.when(g >= _NSBUF)
                    def _():
                        swait(m)

                    scale(u, m, g)
                    sstart(m, g)
                    nb = (u + _NBUF - 1) % _NBUF

                    @pl.when(g + _NBUF - 1 < _SBC)
                    def _():
                        gstart(nb, g + _NBUF - 1)
                return carry2

            lax.fori_loop(0, _SBC // _NBUF, quad_body, 0)
            for m in range(_NSBUF):
                swait(m)
            return carry

        lax.fori_loop(0, _NSB, sb_body, 0)

        plsc.subcore_barrier()

        @pl.when(s < 15)
        def _():
            pltpu.sync_copy(acc.at[pl.ds(r0, stripe)],
                            out.at[pl.ds(c * _NN + r0, stripe)])

        @pl.when(s == 15)
        def _():
            pltpu.sync_copy(acc.at[pl.ds(r0, last)],
                            out.at[pl.ds(c * _NN + r0, last)])

    return spmm


_spmm_hid = _make_spmm(_DHID // 2)
_spmm_out = _make_spmm(_DOUT // 2)


def kernel(edge_index1, edge_weight1, edge_index2, edge_weight2,
           X1, X2, W1, b1, W2, b2):
    x = jnp.concatenate([X1, X2], axis=0)
    row = jnp.concatenate([edge_index1[0], edge_index2[0] + _N])
    col = jnp.concatenate([edge_index1[1], edge_index2[1] + _N])
    w = jnp.concatenate([edge_weight1, edge_weight2])
    pad = _EPAD - _E2
    row = jnp.pad(row, (0, pad)).reshape(_EPADC, _K)
    col = jnp.pad(col, (0, pad)).reshape(_EPADC, _K)
    w = jnp.pad(w, (0, pad))

    w1q = jnp.take(W1, _Q128, axis=1)
    w2q = jnp.take(W2, _Q64, axis=1)
    b1r = b1.reshape(1, _DHID)
    b2r = b2.reshape(1, _DOUT)

    z = _mm1(x, w1q)
    z = _spmm_hid(z.reshape(2 * _NN, _DHID // 2), row, col, w)
    z = _mid(z.reshape(2, _NN, _DHID // 2), b1r, w2q)
    z = _spmm_out(z.reshape(2 * _NN, _DOUT // 2), row, col, w)
    z = _final(z.reshape(2, _NN, _DOUT // 2), b2r)
    return z[:_N], z[_N:]
